# baseline (speedup 1.0000x reference)
import functools

import jax
import jax.numpy as jnp
from jax import lax
from jax.experimental import pallas as pl
from jax.experimental.pallas import tpu as pltpu
from jax.experimental.pallas import tpu_sc as plsc

NC = 2
NS = 16
L = 16


def _body(mu_hbm, sig_hbm, mu_tab_hbm, sig_tab_hbm, idx_hbm,
          out_mu_hbm, out_sig_hbm,
          mu_v, sig_v, idx_v, row_v, tab_v,
          acc_mu, acc_sig, acc_cnt, pstage, cstage,
          sh_mu, sh_sig, sh_cnt,
          *, B, C, D, K):
    CH = C // NC
    rows_per_tile = B // NS
    n_chunks = rows_per_tile // K
    c = lax.axis_index("c")
    s = lax.axis_index("s")
    ch0 = c * CH

    i32 = jnp.int32
    zeros = jnp.zeros((L,), jnp.float32)
    ones = jnp.ones((L,), jnp.float32)
    iota = jax.lax.iota(i32, L)

    def zinit(i, carry):
        acc_mu[pl.ds(i * L, L)] = zeros
        acc_sig[pl.ds(i * L, L)] = zeros
        return carry
    lax.fori_loop(0, D * CH // L, zinit, 0)
    acc_cnt[pl.ds(0, L)] = zeros
    for q in range(1, D * L // L):
        acc_cnt[pl.ds(q * L, L)] = zeros

    def chunk(g, carry):
        base = s * rows_per_tile + g * K
        pltpu.sync_copy(idx_hbm.at[pl.ds(base, K)], idx_v)
        pltpu.sync_copy(mu_hbm.at[pl.ds(base, K), pl.ds(ch0, CH)], mu_v)
        pltpu.sync_copy(sig_hbm.at[pl.ds(base, K), pl.ds(ch0, CH)], sig_v)

        for q in range(K // L):
            idx_l = idx_v[pl.ds(q * L, L)]
            plsc.addupdate_scatter(acc_cnt, [idx_l * L + iota], ones)

        def row(r, carry2):
            dvec = plsc.load_gather(idx_v, [jnp.full((L,), r, i32)])
            addr = dvec * CH + iota
            for j in range(CH // L):
                sl = pl.ds(j * L, L)
                plsc.addupdate_scatter(acc_mu, [addr], mu_v[r, sl])
                plsc.addupdate_scatter(acc_sig, [addr], sig_v[r, sl])
                addr = addr + L
            return carry2
        lax.fori_loop(0, K, row, 0)
        return carry

    lax.fori_loop(0, n_chunks, chunk, 0)

    pltpu.sync_copy(acc_mu, sh_mu.at[s])
    pltpu.sync_copy(acc_sig, sh_sig.at[s])
    pltpu.sync_copy(acc_cnt, sh_cnt.at[s])
    plsc.subcore_barrier()

    pltpu.sync_copy(sh_cnt.at[:, pl.ds(s * L, L)], cstage)
    cvec = cstage[0, :]
    for t in range(1, NS):
        cvec = cvec + cstage[t, :]
    cnt = jnp.sum(cvec)
    csplat = jnp.full((L,), cnt)
    present = csplat > 0.0
    recip = 1.0 / jnp.maximum(csplat, 1.0)

    for arr_sh, arr_tab, arr_out in (
        (sh_mu, mu_tab_hbm, out_mu_hbm),
        (sh_sig, sig_tab_hbm, out_sig_hbm),
    ):
        pltpu.sync_copy(arr_sh.at[:, pl.ds(s * CH, CH)], pstage)
        pltpu.sync_copy(arr_tab.at[s, pl.ds(ch0, CH)], tab_v)
        for j in range(CH // L):
            sl = pl.ds(j * L, L)
            x = pstage[0, sl]
            for t in range(1, NS):
                x = x + pstage[t, sl]
            row_v[sl] = jnp.where(present, x * recip, tab_v[sl])
        pltpu.sync_copy(row_v, arr_out.at[s, pl.ds(ch0, CH)])


@jax.jit
def _style_stats(mu, sig, mu_table, sig_table, domain_idx):
    B, C = mu.shape
    D = mu_table.shape[0]
    K = 64
    CH = C // NC
    assert D == NS and B % (NS * K) == 0 and C % (NC * L) == 0

    mesh = plsc.VectorSubcoreMesh(core_axis_name="c", subcore_axis_name="s")
    f32 = jnp.float32
    kern = pl.kernel(
        functools.partial(_body, B=B, C=C, D=D, K=K),
        out_type=(jax.ShapeDtypeStruct((D, C), f32),
                  jax.ShapeDtypeStruct((D, C), f32)),
        mesh=mesh,
        compiler_params=pltpu.CompilerParams(use_tc_tiling_on_sc=False,
                                              needs_layout_passes=False),
        scratch_types=[
            pltpu.VMEM((K, CH), f32),
            pltpu.VMEM((K, CH), f32),
            pltpu.VMEM((K,), jnp.int32),
            pltpu.VMEM((CH,), f32),
            pltpu.VMEM((CH,), f32),
            pltpu.VMEM((D * CH,), f32),
            pltpu.VMEM((D * CH,), f32),
            pltpu.VMEM((D * L,), f32),
            pltpu.VMEM((NS, CH), f32),
            pltpu.VMEM((NS, L), f32),
            pltpu.VMEM_SHARED((NS, D * CH), f32),
            pltpu.VMEM_SHARED((NS, D * CH), f32),
            pltpu.VMEM_SHARED((NS, D * L), f32),
        ],
    )
    return kern(mu, sig, mu_table, sig_table, domain_idx)


def kernel(mu, sig, mu_table, sig_table, domain_idx, layer_idx):
    del layer_idx
    return _style_stats(mu, sig, mu_table, sig_table, domain_idx)


# speedup vs baseline: 1.1764x; 1.1764x over previous
import functools

import jax
import jax.numpy as jnp
from jax import lax
from jax.experimental import pallas as pl
from jax.experimental.pallas import tpu as pltpu
from jax.experimental.pallas import tpu_sc as plsc

NC = 2
NS = 16
L = 16


def _body(mu_hbm, sig_hbm, mu_tab_hbm, sig_tab_hbm, idx_hbm,
          out_mu_hbm, out_sig_hbm,
          mu_v, sig_v, idx_v, row_v, tab_v,
          acc_mu, acc_sig, acc_cnt, pstage, cstage, sems,
          sh_mu, sh_sig, sh_cnt,
          *, B, C, D, K):
    CH = C // NC
    rows_per_tile = B // NS
    n_chunks = rows_per_tile // K
    c = lax.axis_index("c")
    s = lax.axis_index("s")
    ch0 = c * CH

    i32 = jnp.int32
    zeros = jnp.zeros((L,), jnp.float32)
    ones = jnp.ones((L,), jnp.float32)
    iota = jax.lax.iota(i32, L)

    def zinit(i, carry):
        acc_mu[pl.ds(i * L, L)] = zeros
        acc_sig[pl.ds(i * L, L)] = zeros
        return carry
    lax.fori_loop(0, D * CH // L, zinit, 0)
    acc_cnt[pl.ds(0, L)] = zeros
    for q in range(1, D * L // L):
        acc_cnt[pl.ds(q * L, L)] = zeros

    row0 = s * rows_per_tile

    def issue(g, b):
        base = row0 + g * K
        pltpu.async_copy(idx_hbm.at[pl.ds(base, K)], idx_v.at[b], sems.at[b])
        pltpu.async_copy(mu_hbm.at[pl.ds(base, K), pl.ds(ch0, CH)],
                         mu_v.at[b], sems.at[b])
        pltpu.async_copy(sig_hbm.at[pl.ds(base, K), pl.ds(ch0, CH)],
                         sig_v.at[b], sems.at[b])

    def drain(b):
        pltpu.make_async_copy(idx_hbm.at[pl.ds(0, K)], idx_v.at[b],
                              sems.at[b]).wait()
        pltpu.make_async_copy(mu_hbm.at[pl.ds(0, K), pl.ds(0, CH)],
                              mu_v.at[b], sems.at[b]).wait()
        pltpu.make_async_copy(sig_hbm.at[pl.ds(0, K), pl.ds(0, CH)],
                              sig_v.at[b], sems.at[b]).wait()

    def consume(b):
        for q in range(K // L):
            idx_l = idx_v[b, pl.ds(q * L, L)]
            plsc.addupdate_scatter(acc_cnt, [idx_l * L + iota], ones)

        def row(r, carry2):
            dvec = plsc.load_gather(idx_v.at[b], [jnp.full((L,), r, i32)])
            addr = dvec * CH + iota
            for j in range(CH // L):
                sl = pl.ds(j * L, L)
                plsc.addupdate_scatter(acc_mu, [addr], mu_v[b, r, sl])
                plsc.addupdate_scatter(acc_sig, [addr], sig_v[b, r, sl])
                addr = addr + L
            return carry2
        lax.fori_loop(0, K, row, 0)

    issue(0, 0)
    issue(1, 1)

    @pl.loop(0, n_chunks, step=2)
    def _chunks(g):
        for b in range(2):
            drain(b)
            consume(b)

            @pl.when(g + b + 2 < n_chunks)
            def _():
                issue(g + b + 2, b)

    pltpu.sync_copy(acc_mu, sh_mu.at[s])
    pltpu.sync_copy(acc_sig, sh_sig.at[s])
    pltpu.sync_copy(acc_cnt, sh_cnt.at[s])
    plsc.subcore_barrier()

    pltpu.sync_copy(sh_cnt.at[:, pl.ds(s * L, L)], cstage)
    cvec = cstage[0, :]
    for t in range(1, NS):
        cvec = cvec + cstage[t, :]
    cnt = jnp.sum(cvec)
    csplat = jnp.full((L,), cnt)
    present = csplat > 0.0
    recip = 1.0 / jnp.maximum(csplat, 1.0)

    for arr_sh, arr_tab, arr_out in (
        (sh_mu, mu_tab_hbm, out_mu_hbm),
        (sh_sig, sig_tab_hbm, out_sig_hbm),
    ):
        pltpu.sync_copy(arr_sh.at[:, pl.ds(s * CH, CH)], pstage)
        pltpu.sync_copy(arr_tab.at[s, pl.ds(ch0, CH)], tab_v)
        for j in range(CH // L):
            sl = pl.ds(j * L, L)
            x = pstage[0, sl]
            for t in range(1, NS):
                x = x + pstage[t, sl]
            row_v[sl] = jnp.where(present, x * recip, tab_v[sl])
        pltpu.sync_copy(row_v, arr_out.at[s, pl.ds(ch0, CH)])


@jax.jit
def _style_stats(mu, sig, mu_table, sig_table, domain_idx):
    B, C = mu.shape
    D = mu_table.shape[0]
    K = 32
    CH = C // NC
    assert D == NS and B % (NS * K) == 0 and C % (NC * L) == 0

    mesh = plsc.VectorSubcoreMesh(core_axis_name="c", subcore_axis_name="s")
    f32 = jnp.float32
    kern = pl.kernel(
        functools.partial(_body, B=B, C=C, D=D, K=K),
        out_type=(jax.ShapeDtypeStruct((D, C), f32),
                  jax.ShapeDtypeStruct((D, C), f32)),
        mesh=mesh,
        compiler_params=pltpu.CompilerParams(use_tc_tiling_on_sc=False,
                                              needs_layout_passes=False),
        scratch_types=[
            pltpu.VMEM((2, K, CH), f32),
            pltpu.VMEM((2, K, CH), f32),
            pltpu.VMEM((2, K), jnp.int32),
            pltpu.VMEM((CH,), f32),
            pltpu.VMEM((CH,), f32),
            pltpu.VMEM((D * CH,), f32),
            pltpu.VMEM((D * CH,), f32),
            pltpu.VMEM((D * L,), f32),
            pltpu.VMEM((NS, CH), f32),
            pltpu.VMEM((NS, L), f32),
            pltpu.SemaphoreType.DMA((2,)),
            pltpu.VMEM_SHARED((NS, D * CH), f32),
            pltpu.VMEM_SHARED((NS, D * CH), f32),
            pltpu.VMEM_SHARED((NS, D * L), f32),
        ],
    )
    return kern(mu, sig, mu_table, sig_table, domain_idx)


def kernel(mu, sig, mu_table, sig_table, domain_idx, layer_idx):
    del layer_idx
    return _style_stats(mu, sig, mu_table, sig_table, domain_idx)


# speedup vs baseline: 3.4261x; 2.9123x over previous
import functools

import jax
import jax.numpy as jnp
from jax import lax
from jax.experimental import pallas as pl
from jax.experimental.pallas import tpu as pltpu
from jax.experimental.pallas import tpu_sc as plsc

NC = 2
NS = 16
L = 16
CB = 128


def _body(mu_hbm, sig_hbm, mu_tab_hbm, sig_tab_hbm, idx_hbm,
          out_mu_hbm, out_sig_hbm,
          mu_v, sig_v, idx_v, row_v, st_v, tab_v,
          acc_mu, acc_sig, acc_cnt, sems,
          sh_mu, sh_sig, sh_cnt,
          *, B, C, D, K):
    CH = C // NC
    rows_per_tile = B // NS
    n_chunks = rows_per_tile // K
    c = lax.axis_index("c")
    s = lax.axis_index("s")
    ch0 = c * CH

    i32 = jnp.int32
    zeros = jnp.zeros((L,), jnp.float32)
    ones = jnp.ones((L,), jnp.float32)
    iota = jax.lax.iota(i32, L)

    def zinit(i, carry):
        acc_mu[pl.ds(i * L, L)] = zeros
        acc_sig[pl.ds(i * L, L)] = zeros
        return carry
    lax.fori_loop(0, D * CH // L, zinit, 0)
    for d in range(D):
        acc_cnt[pl.ds(d * CB, L)] = zeros

    pltpu.sync_copy(idx_hbm.at[pl.ds(s * rows_per_tile, rows_per_tile)], idx_v)

    row0 = s * rows_per_tile

    def issue(g, b):
        base = row0 + g * K
        pltpu.async_copy(mu_hbm.at[pl.ds(base, K), pl.ds(ch0, CH)],
                         mu_v.at[b], sems.at[b])
        pltpu.async_copy(sig_hbm.at[pl.ds(base, K), pl.ds(ch0, CH)],
                         sig_v.at[b], sems.at[b])

    def drain(b):
        pltpu.make_async_copy(mu_hbm.at[pl.ds(0, K), pl.ds(0, CH)],
                              mu_v.at[b], sems.at[b]).wait()
        pltpu.make_async_copy(sig_hbm.at[pl.ds(0, K), pl.ds(0, CH)],
                              sig_v.at[b], sems.at[b]).wait()

    G = 4

    def consume(g, b):
        for q in range(K // L):
            idx_l = idx_v[pl.ds(g * K + q * L, L)]
            plsc.addupdate_scatter(acc_cnt, [idx_l * CB + iota], ones)

        def row(r, carry2):
            dvec = plsc.load_gather(idx_v, [jnp.full((L,), g * K + r, i32)])
            base = dvec * CH
            for j0 in range(0, CH // L, G):
                ms = [mu_v[b, r, pl.ds((j0 + t) * L, L)] for t in range(G)]
                ss = [sig_v[b, r, pl.ds((j0 + t) * L, L)] for t in range(G)]
                ads = [base + (iota + (j0 + t) * L) for t in range(G)]
                for t in range(G):
                    plsc.addupdate_scatter(acc_mu, [ads[t]], ms[t])
                    plsc.addupdate_scatter(acc_sig, [ads[t]], ss[t])
            return carry2
        lax.fori_loop(0, K, row, 0)

    issue(0, 0)
    issue(1, 1)

    @pl.loop(0, n_chunks, step=2)
    def _chunks(g):
        for b in range(2):
            drain(b)
            consume(g + b, b)

            @pl.when(g + b + 2 < n_chunks)
            def _():
                issue(g + b + 2, b)

    pltpu.sync_copy(acc_mu, sh_mu.at[pl.ds(s * D * CH, D * CH)])
    pltpu.sync_copy(acc_sig, sh_sig.at[pl.ds(s * D * CH, D * CH)])
    pltpu.sync_copy(acc_cnt, sh_cnt.at[pl.ds(s * D * CB, D * CB)])
    plsc.subcore_barrier()

    for t in range(NS):
        pltpu.sync_copy(sh_cnt.at[pl.ds(t * D * CB + s * CB, L)],
                        st_v.at[pl.ds(t * L, L)])
    cvec = st_v[pl.ds(0, L)]
    for t in range(1, NS):
        cvec = cvec + st_v[pl.ds(t * L, L)]
    cnt = jnp.sum(cvec)
    csplat = jnp.full((L,), cnt)
    present = csplat > 0.0
    recip = 1.0 / jnp.maximum(csplat, 1.0)

    for arr_sh, arr_tab, arr_out in (
        (sh_mu, mu_tab_hbm, out_mu_hbm),
        (sh_sig, sig_tab_hbm, out_sig_hbm),
    ):
        for j in range(CH // L):
            row_v[pl.ds(j * L, L)] = zeros
        for t in range(NS):
            pltpu.sync_copy(arr_sh.at[pl.ds(t * D * CH + s * CH, CH)], tab_v)
            for j in range(CH // L):
                plsc.addupdate(row_v.at[pl.ds(j * L, L)], tab_v[pl.ds(j * L, L)])
        pltpu.sync_copy(arr_tab.at[pl.ds(s * C + ch0, CH)], tab_v)
        for j in range(CH // L):
            sl = pl.ds(j * L, L)
            row_v[sl] = jnp.where(present, row_v[sl] * recip, tab_v[sl])
        pltpu.sync_copy(row_v, arr_out.at[pl.ds(s * C + ch0, CH)])


@jax.jit
def _style_stats(mu, sig, mu_table, sig_table, domain_idx):
    B, C = mu.shape
    D = mu_table.shape[0]
    K = 32
    CH = C // NC
    assert D == NS and B % (NS * K) == 0 and C % (NC * L) == 0

    mesh = plsc.VectorSubcoreMesh(core_axis_name="c", subcore_axis_name="s")
    f32 = jnp.float32
    kern = pl.kernel(
        functools.partial(_body, B=B, C=C, D=D, K=K),
        out_type=(jax.ShapeDtypeStruct((D * C,), f32),
                  jax.ShapeDtypeStruct((D * C,), f32)),
        mesh=mesh,
        compiler_params=pltpu.CompilerParams(needs_layout_passes=False),
        scratch_types=[
            pltpu.VMEM((2, K, CH), f32),
            pltpu.VMEM((2, K, CH), f32),
            pltpu.VMEM((B // NS,), jnp.int32),
            pltpu.VMEM((CH,), f32),
            pltpu.VMEM((NS * L,), f32),
            pltpu.VMEM((CH,), f32),
            pltpu.VMEM((D * CH,), f32),
            pltpu.VMEM((D * CH,), f32),
            pltpu.VMEM((D * CB,), f32),
            pltpu.SemaphoreType.DMA((2,)),
            pltpu.VMEM_SHARED((NS * D * CH,), f32),
            pltpu.VMEM_SHARED((NS * D * CH,), f32),
            pltpu.VMEM_SHARED((NS * D * CB,), f32),
        ],
    )
    out_mu, out_sig = kern(mu, sig, mu_table.reshape(-1),
                           sig_table.reshape(-1), domain_idx)
    return out_mu.reshape(D, C), out_sig.reshape(D, C)


def kernel(mu, sig, mu_table, sig_table, domain_idx, layer_idx):
    del layer_idx
    return _style_stats(mu, sig, mu_table, sig_table, domain_idx)
